# Initial kernel scaffold; baseline (speedup 1.0000x reference)
import jax
import jax.numpy as jnp
from jax.experimental import pallas as pl


def kernel(x, edge_index, edge_attr, W_e0, b_e0, W_n0, b_n0, W_e1, b_e1, W_n1, b_n1, W_e2, b_e2, W_n2, b_n2, W_m0, b_m0, W_m1, b_m1, W_m2, b_m2):
    raise NotImplementedError("write your pallas kernel here")


# speedup vs baseline: 3.2020x; 3.2020x over previous
import functools

import jax
import jax.numpy as jnp
from jax import lax
from jax.experimental import pallas as pl
from jax.experimental.pallas import tpu as pltpu
from jax.experimental.pallas import tpu_sc as plsc

N = 10000
E = 320000
DN = 128
DH = 16

NC = 2
NS = 16
NW = NC * NS
CHUNK = 128
NCHUNK = E // CHUNK
KFULL = NCHUNK // NW
REM = NCHUNK - KFULL * NW
NPAD = 10240
RPT = NPAD // NS
EB = 3200

_f32 = jnp.float32


def _proj_body(x_ref, ws_ref, wd_ref, p_ref, q_ref):
    x = x_ref[...]
    pad = jnp.zeros((NPAD - N, DH), _f32)
    p_ref[...] = jnp.concatenate(
        [jnp.dot(x, ws_ref[...], preferred_element_type=_f32), pad])
    q_ref[...] = jnp.concatenate(
        [jnp.dot(x, wd_ref[...], preferred_element_type=_f32), pad])


def _proj(x, ws, wd):
    return pl.pallas_call(
        _proj_body,
        out_shape=(jax.ShapeDtypeStruct((NPAD, DH), _f32),
                   jax.ShapeDtypeStruct((NPAD, DH), _f32)),
    )(x, ws, wd)


def _edense_body(e_ref, w_ref, b_ref, o_ref):
    o_ref[...] = (jnp.dot(e_ref[...], w_ref[...], preferred_element_type=_f32)
                  + b_ref[...])


def _edense(e, w, b):
    de = e.shape[1]
    return pl.pallas_call(
        _edense_body,
        grid=(E // EB,),
        in_specs=[pl.BlockSpec((EB, de), lambda i: (i, 0)),
                  pl.BlockSpec((de, DH), lambda i: (0, 0)),
                  pl.BlockSpec((1, DH), lambda i: (0, 0))],
        out_specs=pl.BlockSpec((EB, DH), lambda i: (i, 0)),
        out_shape=jax.ShapeDtypeStruct((E, DH), _f32),
    )(e, w, b.reshape(1, DH))


def _edense_avg_body(e0_ref, e1_ref, w_ref, b_ref, o_ref):
    e = 0.5 * (e0_ref[...] + e1_ref[...])
    o_ref[...] = jnp.dot(e, w_ref[...], preferred_element_type=_f32) + b_ref[...]


def _edense_avg(e0, e1, w, b):
    return pl.pallas_call(
        _edense_avg_body,
        grid=(E // EB,),
        in_specs=[pl.BlockSpec((EB, DH), lambda i: (i, 0)),
                  pl.BlockSpec((EB, DH), lambda i: (i, 0)),
                  pl.BlockSpec((DH, DH), lambda i: (0, 0)),
                  pl.BlockSpec((1, DH), lambda i: (0, 0))],
        out_specs=pl.BlockSpec((EB, DH), lambda i: (i, 0)),
        out_shape=jax.ShapeDtypeStruct((E, DH), _f32),
    )(e0, e1, w, b.reshape(1, DH))


def _node_body(x_ref, a_ref, wx_ref, wa_ref, b_ref, o_ref, *, residual):
    a = (a_ref[0] + a_ref[1])[:N]
    h = (jnp.dot(x_ref[...], wx_ref[...], preferred_element_type=_f32)
         + jnp.dot(a, wa_ref[...], preferred_element_type=_f32)
         + b_ref[...])
    h = jnp.maximum(h, 0.0)
    if residual:
        h = 0.5 * (x_ref[...] + h)
    o_ref[...] = h


def _node(x, agg, wx, wa, b, residual):
    return pl.pallas_call(
        functools.partial(_node_body, residual=residual),
        out_shape=jax.ShapeDtypeStruct((N, DN), _f32),
    )(x, agg, wx, wa, b.reshape(1, DN))


def _head_body(e_ref, w0_ref, b0_ref, w1_ref, b1_ref, w2_ref, b2_ref, o_ref):
    h = jnp.maximum(jnp.dot(e_ref[...], w0_ref[...],
                            preferred_element_type=_f32) + b0_ref[...], 0.0)
    h = jnp.maximum(jnp.dot(h, w1_ref[...],
                            preferred_element_type=_f32) + b1_ref[...], 0.0)
    o_ref[...] = jnp.dot(h, w2_ref[...], preferred_element_type=_f32) + b2_ref[...]


def _head(e, w0, b0, w1, b1, w2, b2):
    return pl.pallas_call(
        _head_body,
        grid=(E // EB,),
        in_specs=[pl.BlockSpec((EB, DH), lambda i: (i, 0)),
                  pl.BlockSpec((DH, DH), lambda i: (0, 0)),
                  pl.BlockSpec((1, DH), lambda i: (0, 0)),
                  pl.BlockSpec((DH, DH), lambda i: (0, 0)),
                  pl.BlockSpec((1, DH), lambda i: (0, 0)),
                  pl.BlockSpec((DH, 1), lambda i: (0, 0)),
                  pl.BlockSpec((1, 1), lambda i: (0, 0))],
        out_specs=pl.BlockSpec((EB, 1), lambda i: (i, 0)),
        out_shape=jax.ShapeDtypeStruct((E, 1), _f32),
    )(e, w0, b0.reshape(1, DH), w1, b1.reshape(1, DH), w2, b2.reshape(1, 1))


def _do_chunk(c, p_h, q_h, d_h, s_h, t_h, e_h, sidx, didx, pbuf, qbuf, dbuf,
              ebuf, sem, aggs):
    off = c * CHUNK
    pltpu.sync_copy(s_h.at[c], sidx)
    pltpu.sync_copy(t_h.at[c], didx)
    pltpu.sync_copy(d_h.at[pl.ds(off, CHUNK)], dbuf)
    cp_p = pltpu.async_copy(p_h.at[sidx.at[0]], pbuf, sem)
    cp_q = pltpu.async_copy(q_h.at[didx.at[0]], qbuf, sem)
    cp_p.wait()
    cp_q.wait()

    def rows(g, carry):
        base = g * 16
        for jj in range(16):
            j = base + jj
            ebuf[j] = jnp.maximum(pbuf[j] + qbuf[j] + dbuf[j], 0.0)
        return carry

    lax.fori_loop(0, CHUNK // 16, rows, 0)
    pltpu.sync_copy(ebuf, e_h.at[pl.ds(off, CHUNK)])
    if aggs is not None:
        pltpu.sync_copy(ebuf, aggs.at[didx.at[0]], add=True)


def _sc_edge_loop(p_h, q_h, d_h, s_h, t_h, e_h, sidx, didx, pbuf, qbuf, dbuf,
                  ebuf, sem, aggs):
    cid = lax.axis_index("c")
    sid = lax.axis_index("s")
    wid = sid * NC + cid

    def step(k, carry):
        _do_chunk(wid + k * NW, p_h, q_h, d_h, s_h, t_h, e_h, sidx, didx,
                  pbuf, qbuf, dbuf, ebuf, sem, aggs)
        return carry

    lax.fori_loop(0, KFULL, step, 0)

    @pl.when(wid < REM)
    def _tail():
        _do_chunk(KFULL * NW + wid, p_h, q_h, d_h, s_h, t_h, e_h, sidx, didx,
                  pbuf, qbuf, dbuf, ebuf, sem, aggs)


def _sc_edge_agg_body(p_h, q_h, d_h, s_h, t_h, z_h, e_h, agg_h,
                      sidx, didx, pbuf, qbuf, dbuf, ebuf, sem, aggs):
    cid = lax.axis_index("c")
    sid = lax.axis_index("s")
    rows = pl.ds(sid * RPT, RPT)
    pltpu.sync_copy(z_h.at[rows], aggs.at[rows])
    plsc.subcore_barrier()
    _sc_edge_loop(p_h, q_h, d_h, s_h, t_h, e_h, sidx, didx, pbuf, qbuf, dbuf,
                  ebuf, sem, aggs)
    plsc.subcore_barrier()
    pltpu.sync_copy(aggs.at[rows], agg_h.at[cid, rows])


def _sc_edge_body(p_h, q_h, d_h, s_h, t_h, e_h,
                  sidx, didx, pbuf, qbuf, dbuf, ebuf, sem):
    _sc_edge_loop(p_h, q_h, d_h, s_h, t_h, e_h, sidx, didx, pbuf, qbuf, dbuf,
                  ebuf, sem, None)


_SC_MESH = plsc.VectorSubcoreMesh(core_axis_name="c", subcore_axis_name="s")
_SC_PARAMS = pltpu.CompilerParams(use_tc_tiling_on_sc=False)

_SC_SCRATCH = [
    pltpu.VMEM((1, CHUNK), jnp.int32),
    pltpu.VMEM((1, CHUNK), jnp.int32),
    pltpu.VMEM((CHUNK, DH), _f32),
    pltpu.VMEM((CHUNK, DH), _f32),
    pltpu.VMEM((CHUNK, DH), _f32),
    pltpu.VMEM((CHUNK, DH), _f32),
    pltpu.SemaphoreType.DMA,
]


def _sc_edge_agg(p, q, d, src2, dst2, zeros):
    f = pl.kernel(
        _sc_edge_agg_body,
        out_type=(jax.ShapeDtypeStruct((E, DH), _f32),
                  jax.ShapeDtypeStruct((NC, NPAD, DH), _f32)),
        mesh=_SC_MESH,
        scratch_types=_SC_SCRATCH + [pltpu.VMEM_SHARED((NPAD, DH), _f32)],
        compiler_params=_SC_PARAMS,
    )
    return f(p, q, d, src2, dst2, zeros)


def _sc_edge(p, q, d, src2, dst2):
    f = pl.kernel(
        _sc_edge_body,
        out_type=jax.ShapeDtypeStruct((E, DH), _f32),
        mesh=_SC_MESH,
        scratch_types=_SC_SCRATCH,
        compiler_params=_SC_PARAMS,
    )
    return f(p, q, d, src2, dst2)


def kernel(x, edge_index, edge_attr,
           W_e0, b_e0, W_n0, b_n0,
           W_e1, b_e1, W_n1, b_n1,
           W_e2, b_e2, W_n2, b_n2,
           W_m0, b_m0, W_m1, b_m1, W_m2, b_m2):
    x = x.astype(_f32)
    src2 = edge_index[0].reshape(NCHUNK, 1, CHUNK)
    dst2 = edge_index[1].reshape(NCHUNK, 1, CHUNK)
    zeros = jnp.zeros((NPAD, DH), _f32)

    p0, q0 = _proj(x, W_e0[:DN], W_e0[DN:2 * DN])
    d0 = _edense(edge_attr, W_e0[2 * DN:], b_e0)
    e0, agg0 = _sc_edge_agg(p0, q0, d0, src2, dst2, zeros)
    x1 = _node(x, agg0, W_n0[:DN], W_n0[DN:], b_n0, residual=False)

    p1, q1 = _proj(x1, W_e1[:DN], W_e1[DN:2 * DN])
    d1 = _edense(e0, W_e1[2 * DN:], b_e1)
    e1, agg1 = _sc_edge_agg(p1, q1, d1, src2, dst2, zeros)
    x2 = _node(x1, agg1, W_n1[:DN], W_n1[DN:], b_n1, residual=True)

    p2, q2 = _proj(x2, W_e2[:DN], W_e2[DN:2 * DN])
    d2 = _edense_avg(e0, e1, W_e2[2 * DN:], b_e2)
    e2 = _sc_edge(p2, q2, d2, src2, dst2)

    out = _head(e2, W_m0, b_m0, W_m1, b_m1, W_m2, b_m2)
    return jnp.reshape(out, (E,))


# speedup vs baseline: 4.1604x; 1.2993x over previous
import functools

import jax
import jax.numpy as jnp
from jax import lax
from jax.experimental import pallas as pl
from jax.experimental.pallas import tpu as pltpu
from jax.experimental.pallas import tpu_sc as plsc

N = 10000
E = 320000
DN = 128
DH = 16

NC = 2
NS = 16
NW = NC * NS
CHUNK = 128
NCHUNK = E // CHUNK
B4 = (NCHUNK // NW) // 4 * 4
QREM = (NCHUNK - NW * B4) // 4
KMAX = B4 + 4
NCHUNKP = NCHUNK + 4
NPAD = 10240
RPT = NPAD // NS
EB = 3200

_f32 = jnp.float32


def _proj_body(x_ref, ws_ref, wd_ref, p_ref, q_ref):
    x = x_ref[...]
    pad = jnp.zeros((NPAD - N, DH), _f32)
    p_ref[...] = jnp.concatenate(
        [jnp.dot(x, ws_ref[...], preferred_element_type=_f32), pad])
    q_ref[...] = jnp.concatenate(
        [jnp.dot(x, wd_ref[...], preferred_element_type=_f32), pad])


def _proj(x, ws, wd):
    return pl.pallas_call(
        _proj_body,
        out_shape=(jax.ShapeDtypeStruct((NPAD, DH), _f32),
                   jax.ShapeDtypeStruct((NPAD, DH), _f32)),
    )(x, ws, wd)


def _edense_body(e_ref, w_ref, b_ref, o_ref):
    o_ref[...] = (jnp.dot(e_ref[...], w_ref[...], preferred_element_type=_f32)
                  + b_ref[...])


def _edense(e, w, b):
    de = e.shape[1]
    return pl.pallas_call(
        _edense_body,
        grid=(E // EB,),
        in_specs=[pl.BlockSpec((EB, de), lambda i: (i, 0)),
                  pl.BlockSpec((de, DH), lambda i: (0, 0)),
                  pl.BlockSpec((1, DH), lambda i: (0, 0))],
        out_specs=pl.BlockSpec((EB, DH), lambda i: (i, 0)),
        out_shape=jax.ShapeDtypeStruct((E, DH), _f32),
    )(e, w, b.reshape(1, DH))


def _edense_avg_body(e0_ref, e1_ref, w_ref, b_ref, o_ref):
    e = 0.5 * (e0_ref[...] + e1_ref[...])
    o_ref[...] = jnp.dot(e, w_ref[...], preferred_element_type=_f32) + b_ref[...]


def _edense_avg(e0, e1, w, b):
    return pl.pallas_call(
        _edense_avg_body,
        grid=(E // EB,),
        in_specs=[pl.BlockSpec((EB, DH), lambda i: (i, 0)),
                  pl.BlockSpec((EB, DH), lambda i: (i, 0)),
                  pl.BlockSpec((DH, DH), lambda i: (0, 0)),
                  pl.BlockSpec((1, DH), lambda i: (0, 0))],
        out_specs=pl.BlockSpec((EB, DH), lambda i: (i, 0)),
        out_shape=jax.ShapeDtypeStruct((E, DH), _f32),
    )(e0, e1, w, b.reshape(1, DH))


def _node_body(x_ref, a_ref, wx_ref, wa_ref, b_ref, o_ref, *, residual):
    a = (a_ref[0] + a_ref[1])[:N]
    h = (jnp.dot(x_ref[...], wx_ref[...], preferred_element_type=_f32)
         + jnp.dot(a, wa_ref[...], preferred_element_type=_f32)
         + b_ref[...])
    h = jnp.maximum(h, 0.0)
    if residual:
        h = 0.5 * (x_ref[...] + h)
    o_ref[...] = h


def _node(x, agg, wx, wa, b, residual):
    return pl.pallas_call(
        functools.partial(_node_body, residual=residual),
        out_shape=jax.ShapeDtypeStruct((N, DN), _f32),
    )(x, agg, wx, wa, b.reshape(1, DN))


def _head_body(e_ref, w0_ref, b0_ref, w1_ref, b1_ref, w2_ref, b2_ref, o_ref):
    h = jnp.maximum(jnp.dot(e_ref[...], w0_ref[...],
                            preferred_element_type=_f32) + b0_ref[...], 0.0)
    h = jnp.maximum(jnp.dot(h, w1_ref[...],
                            preferred_element_type=_f32) + b1_ref[...], 0.0)
    o_ref[...] = jnp.dot(h, w2_ref[...], preferred_element_type=_f32) + b2_ref[...]


def _head(e, w0, b0, w1, b1, w2, b2):
    return pl.pallas_call(
        _head_body,
        grid=(E // EB,),
        in_specs=[pl.BlockSpec((EB, DH), lambda i: (i, 0)),
                  pl.BlockSpec((DH, DH), lambda i: (0, 0)),
                  pl.BlockSpec((1, DH), lambda i: (0, 0)),
                  pl.BlockSpec((DH, DH), lambda i: (0, 0)),
                  pl.BlockSpec((1, DH), lambda i: (0, 0)),
                  pl.BlockSpec((DH, 1), lambda i: (0, 0)),
                  pl.BlockSpec((1, 1), lambda i: (0, 0))],
        out_specs=pl.BlockSpec((EB, 1), lambda i: (i, 0)),
        out_shape=jax.ShapeDtypeStruct((E, 1), _f32),
    )(e, w0, b0.reshape(1, DH), w1, b1.reshape(1, DH), w2, b2.reshape(1, 1))


def _sc_edge_loop(p_h, q_h, d_h, s_h, t_h, e_h, bufs, semis, semos, aggs):
    cid = lax.axis_index("c")
    sid = lax.axis_index("s")
    wid = sid * NC + cid
    c0 = wid * B4 + 4 * jnp.minimum(wid, QREM)
    nq = B4 // 4 + jnp.where(wid < QREM, 1, 0)

    def compute(pb, qb, db, eb):
        def rows(g, carry):
            base = g * 16
            for jj in range(16):
                j = base + jj
                eb[j] = jnp.maximum(pb[j] + qb[j] + db[j], 0.0)
            return carry
        lax.fori_loop(0, CHUNK // 16, rows, 0)

    def quad(jq, carry):
        kb = c0 + 4 * jq
        ins = []
        for i in range(4):
            c = kb + i
            off = c * CHUNK
            pb, qb, db, eb, sb, tb = bufs[i]
            ins.append((
                pltpu.async_copy(d_h.at[pl.ds(off, CHUNK)], db, semis[i]),
                pltpu.async_copy(s_h.at[pl.ds(c, 1)], sb, semis[i]),
                pltpu.async_copy(t_h.at[pl.ds(c, 1)], tb, semis[i]),
            ))
        gath = []
        for i in range(4):
            pb, qb, db, eb, sb, tb = bufs[i]
            for cp in ins[i]:
                cp.wait()
            gath.append((
                pltpu.async_copy(p_h.at[sb.at[0]], pb, semis[i]),
                pltpu.async_copy(q_h.at[tb.at[0]], qb, semis[i]),
            ))
        outs = []
        for i in range(4):
            off = (kb + i) * CHUNK
            pb, qb, db, eb, sb, tb = bufs[i]
            for cp in gath[i]:
                cp.wait()
            compute(pb, qb, db, eb)
            outs.append(pltpu.async_copy(eb, e_h.at[pl.ds(off, CHUNK)],
                                         semos[i]))
            if aggs is not None:
                pltpu.sync_copy(eb, aggs.at[tb.at[0]], add=True)
        for cp in outs:
            cp.wait()
        return carry

    lax.fori_loop(0, nq, quad, 0)


def _sc_edge_agg_body(p_h, q_h, d_h, s_h, t_h, z_h, e_h, agg_h, *refs):
    scratch, aggs = refs[:-1], refs[-1]
    cid = lax.axis_index("c")
    sid = lax.axis_index("s")
    rows = pl.ds(sid * RPT, RPT)
    pltpu.sync_copy(z_h.at[rows], aggs.at[rows])
    plsc.subcore_barrier()
    bufs = [scratch[6 * i:6 * i + 6] for i in range(4)]
    _sc_edge_loop(p_h, q_h, d_h, s_h, t_h, e_h,
                  bufs, scratch[24:28], scratch[28:32], aggs)
    plsc.subcore_barrier()
    pltpu.sync_copy(aggs.at[rows], agg_h.at[cid, rows])


def _sc_edge_body(p_h, q_h, d_h, s_h, t_h, e_h, *scratch):
    bufs = [scratch[6 * i:6 * i + 6] for i in range(4)]
    _sc_edge_loop(p_h, q_h, d_h, s_h, t_h, e_h,
                  bufs, scratch[24:28], scratch[28:32], None)


_SC_MESH = plsc.VectorSubcoreMesh(core_axis_name="c", subcore_axis_name="s")
_SC_PARAMS = pltpu.CompilerParams(use_tc_tiling_on_sc=False)

_SC_SCRATCH = (
    ([pltpu.VMEM((CHUNK, DH), _f32)] * 4
     + [pltpu.VMEM((1, CHUNK), jnp.int32)] * 2) * 4
    + [pltpu.SemaphoreType.DMA] * 8
)


def _sc_edge_agg(p, q, d, src2, dst2, zeros):
    f = pl.kernel(
        _sc_edge_agg_body,
        out_type=(jax.ShapeDtypeStruct((E, DH), _f32),
                  jax.ShapeDtypeStruct((NC, NPAD, DH), _f32)),
        mesh=_SC_MESH,
        scratch_types=_SC_SCRATCH + [pltpu.VMEM_SHARED((NPAD, DH), _f32)],
        compiler_params=_SC_PARAMS,
    )
    return f(p, q, d, src2, dst2, zeros)


def _sc_edge(p, q, d, src2, dst2):
    f = pl.kernel(
        _sc_edge_body,
        out_type=jax.ShapeDtypeStruct((E, DH), _f32),
        mesh=_SC_MESH,
        scratch_types=_SC_SCRATCH,
        compiler_params=_SC_PARAMS,
    )
    return f(p, q, d, src2, dst2)


def kernel(x, edge_index, edge_attr,
           W_e0, b_e0, W_n0, b_n0,
           W_e1, b_e1, W_n1, b_n1,
           W_e2, b_e2, W_n2, b_n2,
           W_m0, b_m0, W_m1, b_m1, W_m2, b_m2):
    x = x.astype(_f32)
    src2 = jnp.pad(edge_index[0].reshape(NCHUNK, CHUNK),
                   ((0, NCHUNKP - NCHUNK), (0, 0)))
    dst2 = jnp.pad(edge_index[1].reshape(NCHUNK, CHUNK),
                   ((0, NCHUNKP - NCHUNK), (0, 0)))
    zeros = jnp.zeros((NPAD, DH), _f32)

    p0, q0 = _proj(x, W_e0[:DN], W_e0[DN:2 * DN])
    d0 = _edense(edge_attr, W_e0[2 * DN:], b_e0)
    e0, agg0 = _sc_edge_agg(p0, q0, d0, src2, dst2, zeros)
    x1 = _node(x, agg0, W_n0[:DN], W_n0[DN:], b_n0, residual=False)

    p1, q1 = _proj(x1, W_e1[:DN], W_e1[DN:2 * DN])
    d1 = _edense(e0, W_e1[2 * DN:], b_e1)
    e1, agg1 = _sc_edge_agg(p1, q1, d1, src2, dst2, zeros)
    x2 = _node(x1, agg1, W_n1[:DN], W_n1[DN:], b_n1, residual=True)

    p2, q2 = _proj(x2, W_e2[:DN], W_e2[DN:2 * DN])
    d2 = _edense_avg(e0, e1, W_e2[2 * DN:], b_e2)
    e2 = _sc_edge(p2, q2, d2, src2, dst2)

    out = _head(e2, W_m0, b_m0, W_m1, b_m1, W_m2, b_m2)
    return jnp.reshape(out, (E,))


# speedup vs baseline: 12.4525x; 2.9931x over previous
import functools

import jax
import jax.numpy as jnp
from jax import lax
from jax.experimental import pallas as pl
from jax.experimental.pallas import tpu as pltpu
from jax.experimental.pallas import tpu_sc as plsc

N = 10000
E = 320000
DN = 128
DH = 16

NC = 2
NS = 16
NW = NC * NS
CHUNK = 128
NCHUNK = E // CHUNK
B4 = (NCHUNK // NW) // 4 * 4
QREM = (NCHUNK - NW * B4) // 4
KMAX = B4 + 4
NCHUNKP = NCHUNK + 4
NPAD = 10240
RPT = NPAD // NS
EP = E // 8
EBP = 4000

_f32 = jnp.float32


def _proj_body(x_ref, ws_ref, wd_ref, p_ref, q_ref):
    x = x_ref[...]
    pad = jnp.zeros((NPAD - N, DH), _f32)
    p_ref[...] = jnp.concatenate(
        [jnp.dot(x, ws_ref[...], preferred_element_type=_f32), pad])
    q_ref[...] = jnp.concatenate(
        [jnp.dot(x, wd_ref[...], preferred_element_type=_f32), pad])


def _proj(x, ws, wd):
    return pl.pallas_call(
        _proj_body,
        out_shape=(jax.ShapeDtypeStruct((NPAD, DH), _f32),
                   jax.ShapeDtypeStruct((NPAD, DH), _f32)),
    )(x, ws, wd)


def _pk(w):
    return jnp.kron(jnp.eye(8, dtype=_f32), w.astype(_f32))


def _pb(b):
    return jnp.tile(b.reshape(1, -1), (1, 8))


def _edense_body(e_ref, w_ref, b_ref, o_ref):
    o_ref[...] = (jnp.dot(e_ref[...], w_ref[...], preferred_element_type=_f32)
                  + b_ref[...])


def _edense(e_p, w, b):
    return pl.pallas_call(
        _edense_body,
        grid=(EP // EBP,),
        in_specs=[pl.BlockSpec((EBP, 128), lambda i: (i, 0)),
                  pl.BlockSpec((128, 128), lambda i: (0, 0)),
                  pl.BlockSpec((1, 128), lambda i: (0, 0))],
        out_specs=pl.BlockSpec((EBP, 128), lambda i: (i, 0)),
        out_shape=jax.ShapeDtypeStruct((EP, 128), _f32),
    )(e_p, _pk(w), _pb(b))


def _edense_avg_body(e0_ref, e1_ref, w_ref, b_ref, o_ref):
    e = 0.5 * (e0_ref[...] + e1_ref[...])
    o_ref[...] = jnp.dot(e, w_ref[...], preferred_element_type=_f32) + b_ref[...]


def _edense_avg(e0_p, e1_p, w, b):
    return pl.pallas_call(
        _edense_avg_body,
        grid=(EP // EBP,),
        in_specs=[pl.BlockSpec((EBP, 128), lambda i: (i, 0)),
                  pl.BlockSpec((EBP, 128), lambda i: (i, 0)),
                  pl.BlockSpec((128, 128), lambda i: (0, 0)),
                  pl.BlockSpec((1, 128), lambda i: (0, 0))],
        out_specs=pl.BlockSpec((EBP, 128), lambda i: (i, 0)),
        out_shape=jax.ShapeDtypeStruct((EP, 128), _f32),
    )(e0_p, e1_p, _pk(w), _pb(b))


def _node_body(x_ref, a_ref, wx_ref, wa_ref, b_ref, o_ref, *, residual):
    a = (a_ref[0] + a_ref[1])[:N]
    h = (jnp.dot(x_ref[...], wx_ref[...], preferred_element_type=_f32)
         + jnp.dot(a, wa_ref[...], preferred_element_type=_f32)
         + b_ref[...])
    h = jnp.maximum(h, 0.0)
    if residual:
        h = 0.5 * (x_ref[...] + h)
    o_ref[...] = h


def _node(x, agg, wx, wa, b, residual):
    return pl.pallas_call(
        functools.partial(_node_body, residual=residual),
        out_shape=jax.ShapeDtypeStruct((N, DN), _f32),
    )(x, agg, wx, wa, b.reshape(1, DN))


def _head_body(e_ref, w0_ref, b0_ref, w1_ref, b1_ref, w2_ref, b2_ref, o_ref):
    h = jnp.maximum(jnp.dot(e_ref[...], w0_ref[...],
                            preferred_element_type=_f32) + b0_ref[...], 0.0)
    h = jnp.maximum(jnp.dot(h, w1_ref[...],
                            preferred_element_type=_f32) + b1_ref[...], 0.0)
    o_ref[...] = jnp.dot(h, w2_ref[...], preferred_element_type=_f32) + b2_ref[...]


def _head(e_p, w0, b0, w1, b1, w2, b2):
    return pl.pallas_call(
        _head_body,
        grid=(EP // EBP,),
        in_specs=[pl.BlockSpec((EBP, 128), lambda i: (i, 0)),
                  pl.BlockSpec((128, 128), lambda i: (0, 0)),
                  pl.BlockSpec((1, 128), lambda i: (0, 0)),
                  pl.BlockSpec((128, 128), lambda i: (0, 0)),
                  pl.BlockSpec((1, 128), lambda i: (0, 0)),
                  pl.BlockSpec((128, 8), lambda i: (0, 0)),
                  pl.BlockSpec((1, 8), lambda i: (0, 0))],
        out_specs=pl.BlockSpec((EBP, 8), lambda i: (i, 0)),
        out_shape=jax.ShapeDtypeStruct((EP, 8), _f32),
    )(e_p, _pk(w0), _pb(b0), _pk(w1), _pb(b1), _pk(w2), _pb(b2))


def _sc_edge_loop(p_h, q_h, d_h, s_h, t_h, e_h, bufs, semis, semos, aggs):
    cid = lax.axis_index("c")
    sid = lax.axis_index("s")
    wid = sid * NC + cid
    c0 = wid * B4 + 4 * jnp.minimum(wid, QREM)
    nq = B4 // 4 + jnp.where(wid < QREM, 1, 0)

    def compute(pb, qb, db, eb):
        def rows(g, carry):
            base = g * 16
            for jj in range(16):
                j = base + jj
                eb[j] = jnp.maximum(pb[j] + qb[j] + db[j], 0.0)
            return carry
        lax.fori_loop(0, CHUNK // 16, rows, 0)

    def quad(jq, carry):
        kb = c0 + 4 * jq
        ins = []
        for i in range(4):
            c = kb + i
            off = c * CHUNK
            pb, qb, db, eb, sb, tb = bufs[i]
            ins.append((
                pltpu.async_copy(d_h.at[pl.ds(off, CHUNK)], db, semis[i]),
                pltpu.async_copy(s_h.at[pl.ds(c, 1)], sb, semis[i]),
                pltpu.async_copy(t_h.at[pl.ds(c, 1)], tb, semis[i]),
            ))
        gath = []
        for i in range(4):
            pb, qb, db, eb, sb, tb = bufs[i]
            for cp in ins[i]:
                cp.wait()
            gath.append((
                pltpu.async_copy(p_h.at[sb.at[0]], pb, semis[i]),
                pltpu.async_copy(q_h.at[tb.at[0]], qb, semis[i]),
            ))
        outs = []
        for i in range(4):
            off = (kb + i) * CHUNK
            pb, qb, db, eb, sb, tb = bufs[i]
            for cp in gath[i]:
                cp.wait()
            compute(pb, qb, db, eb)
            outs.append(pltpu.async_copy(eb, e_h.at[pl.ds(off, CHUNK)],
                                         semos[i]))
            if aggs is not None:
                pltpu.sync_copy(eb, aggs.at[tb.at[0]], add=True)
        for cp in outs:
            cp.wait()
        return carry

    lax.fori_loop(0, nq, quad, 0)


def _sc_edge_agg_body(p_h, q_h, d_h, s_h, t_h, z_h, e_h, agg_h, *refs):
    scratch, aggs = refs[:-1], refs[-1]
    cid = lax.axis_index("c")
    sid = lax.axis_index("s")
    rows = pl.ds(sid * RPT, RPT)
    pltpu.sync_copy(z_h.at[rows], aggs.at[rows])
    plsc.subcore_barrier()
    bufs = [scratch[6 * i:6 * i + 6] for i in range(4)]
    _sc_edge_loop(p_h, q_h, d_h, s_h, t_h, e_h,
                  bufs, scratch[24:28], scratch[28:32], aggs)
    plsc.subcore_barrier()
    pltpu.sync_copy(aggs.at[rows], agg_h.at[cid, rows])


def _sc_edge_body(p_h, q_h, d_h, s_h, t_h, e_h, *scratch):
    bufs = [scratch[6 * i:6 * i + 6] for i in range(4)]
    _sc_edge_loop(p_h, q_h, d_h, s_h, t_h, e_h,
                  bufs, scratch[24:28], scratch[28:32], None)


_SC_MESH = plsc.VectorSubcoreMesh(core_axis_name="c", subcore_axis_name="s")
_SC_PARAMS = pltpu.CompilerParams(use_tc_tiling_on_sc=False)

_SC_SCRATCH = (
    ([pltpu.VMEM((CHUNK, DH), _f32)] * 4
     + [pltpu.VMEM((1, CHUNK), jnp.int32)] * 2) * 4
    + [pltpu.SemaphoreType.DMA] * 8
)


def _sc_edge_agg(p, q, d, src2, dst2, zeros):
    f = pl.kernel(
        _sc_edge_agg_body,
        out_type=(jax.ShapeDtypeStruct((E, DH), _f32),
                  jax.ShapeDtypeStruct((NC, NPAD, DH), _f32)),
        mesh=_SC_MESH,
        scratch_types=_SC_SCRATCH + [pltpu.VMEM_SHARED((NPAD, DH), _f32)],
        compiler_params=_SC_PARAMS,
    )
    return f(p, q, d, src2, dst2, zeros)


def _sc_edge(p, q, d, src2, dst2):
    f = pl.kernel(
        _sc_edge_body,
        out_type=jax.ShapeDtypeStruct((E, DH), _f32),
        mesh=_SC_MESH,
        scratch_types=_SC_SCRATCH,
        compiler_params=_SC_PARAMS,
    )
    return f(p, q, d, src2, dst2)


def kernel(x, edge_index, edge_attr,
           W_e0, b_e0, W_n0, b_n0,
           W_e1, b_e1, W_n1, b_n1,
           W_e2, b_e2, W_n2, b_n2,
           W_m0, b_m0, W_m1, b_m1, W_m2, b_m2):
    x = x.astype(_f32)
    src2 = jnp.pad(edge_index[0].reshape(NCHUNK, CHUNK),
                   ((0, NCHUNKP - NCHUNK), (0, 0)))
    dst2 = jnp.pad(edge_index[1].reshape(NCHUNK, CHUNK),
                   ((0, NCHUNKP - NCHUNK), (0, 0)))
    zeros = jnp.zeros((NPAD, DH), _f32)

    p0, q0 = _proj(x, W_e0[:DN], W_e0[DN:2 * DN])
    d0 = _edense(edge_attr.astype(_f32).reshape(EP, 128), W_e0[2 * DN:], b_e0)
    e0, agg0 = _sc_edge_agg(p0, q0, d0.reshape(E, DH), src2, dst2, zeros)
    x1 = _node(x, agg0, W_n0[:DN], W_n0[DN:], b_n0, residual=False)
    e0_p = e0.reshape(EP, 128)

    p1, q1 = _proj(x1, W_e1[:DN], W_e1[DN:2 * DN])
    d1 = _edense(e0_p, W_e1[2 * DN:], b_e1)
    e1, agg1 = _sc_edge_agg(p1, q1, d1.reshape(E, DH), src2, dst2, zeros)
    x2 = _node(x1, agg1, W_n1[:DN], W_n1[DN:], b_n1, residual=True)

    p2, q2 = _proj(x2, W_e2[:DN], W_e2[DN:2 * DN])
    d2 = _edense_avg(e0_p, e1.reshape(EP, 128), W_e2[2 * DN:], b_e2)
    e2 = _sc_edge(p2, q2, d2.reshape(E, DH), src2, dst2)

    out = _head(e2.reshape(EP, 128), W_m0, b_m0, W_m1, b_m1, W_m2, b_m2)
    return jnp.reshape(out, (E,))
